# Initial kernel scaffold; baseline (speedup 1.0000x reference)
import jax
import jax.numpy as jnp
from jax.experimental import pallas as pl


def kernel(x, weight, bias, gamma, beta):
    raise NotImplementedError("write your pallas kernel here")


# speedup vs baseline: 57.6116x; 57.6116x over previous
import jax
import jax.numpy as jnp
from jax.experimental import pallas as pl
from jax.experimental.pallas import tpu as pltpu

_EPS = 1e-5


def _conv_stats_kernel(w_ref, x_ref, y_ref, s_ref, q_ref):
    _, c, h, w = x_ref.shape
    cout = y_ref.shape[1]
    x = x_ref[0]
    zc = jnp.zeros((c, h, 1), jnp.float32)
    xp = jnp.concatenate([zc, x, zc], axis=2)
    zr = jnp.zeros((c, 1, w + 2), jnp.float32)
    xp = jnp.concatenate([zr, xp, zr], axis=1)

    ys = []
    for co in range(cout):
        acc = None
        for ci in range(c):
            for kh in range(3):
                for kw in range(3):
                    coeff = w_ref[((co * c + ci) * 3 + kh) * 3 + kw]
                    t = xp[ci, kh:kh + h, kw:kw + w] * coeff
                    acc = t if acc is None else acc + t
        ys.append(acc)
    y = jnp.stack(ys, axis=0)
    y_ref[0] = y
    s = jnp.sum(y, axis=(1, 2))
    q = jnp.sum(y * y, axis=(1, 2))
    s_ref[0] = jnp.broadcast_to(s[:, None], (cout, 128))
    q_ref[0] = jnp.broadcast_to(q[:, None], (cout, 128))


def _bn_apply_kernel(sc_ref, sh_ref, y_ref, o_ref):
    cout = o_ref.shape[1]
    for co in range(cout):
        o_ref[0, co] = y_ref[0, co] * sc_ref[co] + sh_ref[co]


def kernel(x, weight, bias, gamma, beta):
    del bias
    n, c, h, w = x.shape
    cout = weight.shape[0]
    m = n * h * w
    wf = weight.astype(jnp.float32).reshape(cout * c * 9)

    y, s_p, q_p = pl.pallas_call(
        _conv_stats_kernel,
        grid=(n,),
        in_specs=[
            pl.BlockSpec(memory_space=pltpu.SMEM),
            pl.BlockSpec((1, c, h, w), lambda i: (i, 0, 0, 0)),
        ],
        out_specs=[
            pl.BlockSpec((1, cout, h, w), lambda i: (i, 0, 0, 0)),
            pl.BlockSpec((1, cout, 128), lambda i: (i, 0, 0)),
            pl.BlockSpec((1, cout, 128), lambda i: (i, 0, 0)),
        ],
        out_shape=(
            jax.ShapeDtypeStruct((n, cout, h, w), jnp.float32),
            jax.ShapeDtypeStruct((n, cout, 128), jnp.float32),
            jax.ShapeDtypeStruct((n, cout, 128), jnp.float32),
        ),
        compiler_params=pltpu.CompilerParams(
            dimension_semantics=("parallel",)),
    )(wf, x)

    s = jnp.sum(s_p[:, :, 0], axis=0)
    q = jnp.sum(q_p[:, :, 0], axis=0)
    mean = s / m
    var = jnp.maximum(q / m - mean * mean, 0.0)
    inv_std = jax.lax.rsqrt(var + jnp.float32(_EPS))
    scale = gamma.astype(jnp.float32) * inv_std
    shift = beta.astype(jnp.float32) - mean * scale

    out = pl.pallas_call(
        _bn_apply_kernel,
        grid=(n,),
        in_specs=[
            pl.BlockSpec(memory_space=pltpu.SMEM),
            pl.BlockSpec(memory_space=pltpu.SMEM),
            pl.BlockSpec((1, cout, h, w), lambda i: (i, 0, 0, 0)),
        ],
        out_specs=pl.BlockSpec((1, cout, h, w), lambda i: (i, 0, 0, 0)),
        out_shape=jax.ShapeDtypeStruct((n, cout, h, w), jnp.float32),
        compiler_params=pltpu.CompilerParams(
            dimension_semantics=("parallel",)),
    )(scale, shift, y)
    return out


# speedup vs baseline: 125.7567x; 2.1828x over previous
import jax
import jax.numpy as jnp
from jax.experimental import pallas as pl
from jax.experimental.pallas import tpu as pltpu

_EPS = 1e-5
_IMGS_PER_BLOCK = 8


def _conv_stats_kernel(w_ref, x_ref, y_ref, s_ref, q_ref):
    b, c, h, w = x_ref.shape
    cout = y_ref.shape[1]
    zc1 = jnp.zeros((h, 1), jnp.float32)
    zr1 = jnp.zeros((1, w), jnp.float32)

    s_tot = [None] * cout
    q_tot = [None] * cout
    for img in range(b):
        planes = []
        for ci in range(c):
            xc = x_ref[img, ci]
            planes.append([
                jnp.concatenate([zc1, xc[:, :w - 1]], axis=1),
                xc,
                jnp.concatenate([xc[:, 1:], zc1], axis=1),
            ])
        for co in range(cout):
            svs = []
            for kh in range(3):
                acc = None
                for ci in range(c):
                    for kw in range(3):
                        coeff = w_ref[((co * c + ci) * 3 + kh) * 3 + kw]
                        t = planes[ci][kw] * coeff
                        acc = t if acc is None else acc + t
                svs.append(acc)
            yc = (jnp.concatenate([zr1, svs[0][:h - 1]], axis=0) + svs[1]
                  + jnp.concatenate([svs[2][1:], zr1], axis=0))
            y_ref[img, co] = yc.astype(y_ref.dtype)
            s = jnp.sum(yc)
            q = jnp.sum(yc * yc)
            s_tot[co] = s if s_tot[co] is None else s_tot[co] + s
            q_tot[co] = q if q_tot[co] is None else q_tot[co] + q

    s_vec = jnp.stack([s_tot[co] for co in range(cout)])
    q_vec = jnp.stack([q_tot[co] for co in range(cout)])
    s_ref[0] = jnp.broadcast_to(s_vec[:, None], (cout, 128))
    q_ref[0] = jnp.broadcast_to(q_vec[:, None], (cout, 128))


def _bn_apply_kernel(sc_ref, sh_ref, y_ref, o_ref):
    b, cout = o_ref.shape[0], o_ref.shape[1]
    for img in range(b):
        for co in range(cout):
            o_ref[img, co] = (y_ref[img, co].astype(jnp.float32)
                              * sc_ref[co] + sh_ref[co])


def kernel(x, weight, bias, gamma, beta):
    del bias
    n, c, h, w = x.shape
    cout = weight.shape[0]
    m = n * h * w
    blk = _IMGS_PER_BLOCK if n % _IMGS_PER_BLOCK == 0 else 1
    nblk = n // blk
    wf = weight.astype(jnp.float32).reshape(cout * c * 9)

    y, s_p, q_p = pl.pallas_call(
        _conv_stats_kernel,
        grid=(nblk,),
        in_specs=[
            pl.BlockSpec(memory_space=pltpu.SMEM),
            pl.BlockSpec((blk, c, h, w), lambda i: (i, 0, 0, 0)),
        ],
        out_specs=[
            pl.BlockSpec((blk, cout, h, w), lambda i: (i, 0, 0, 0)),
            pl.BlockSpec((1, cout, 128), lambda i: (i, 0, 0)),
            pl.BlockSpec((1, cout, 128), lambda i: (i, 0, 0)),
        ],
        out_shape=(
            jax.ShapeDtypeStruct((n, cout, h, w), jnp.bfloat16),
            jax.ShapeDtypeStruct((nblk, cout, 128), jnp.float32),
            jax.ShapeDtypeStruct((nblk, cout, 128), jnp.float32),
        ),
        compiler_params=pltpu.CompilerParams(
            dimension_semantics=("parallel",)),
    )(wf, x)

    s = jnp.sum(s_p[:, :, 0], axis=0)
    q = jnp.sum(q_p[:, :, 0], axis=0)
    mean = s / m
    var = jnp.maximum(q / m - mean * mean, 0.0)
    inv_std = jax.lax.rsqrt(var + jnp.float32(_EPS))
    scale = gamma.astype(jnp.float32) * inv_std
    shift = beta.astype(jnp.float32) - mean * scale

    out = pl.pallas_call(
        _bn_apply_kernel,
        grid=(nblk,),
        in_specs=[
            pl.BlockSpec(memory_space=pltpu.SMEM),
            pl.BlockSpec(memory_space=pltpu.SMEM),
            pl.BlockSpec((blk, cout, h, w), lambda i: (i, 0, 0, 0)),
        ],
        out_specs=pl.BlockSpec((blk, cout, h, w), lambda i: (i, 0, 0, 0)),
        out_shape=jax.ShapeDtypeStruct((n, cout, h, w), jnp.float32),
        compiler_params=pltpu.CompilerParams(
            dimension_semantics=("parallel",)),
    )(scale, shift, y)
    return out
